# baseline (speedup 1.0000x reference)
import functools

import jax
import jax.numpy as jnp
from jax import lax
from jax.experimental import pallas as pl
from jax.experimental.pallas import tpu as pltpu
from jax.experimental.pallas import tpu_sc as plsc

_R, _S = 16384, 26
_V, _D = 10000, 64
_NC, _NS = 2, 16
_NW = _NC * _NS
_ROWS_W = _R // _NW
_ROWS_C = 8
_N_CHUNKS = _ROWS_W // _ROWS_C
_N_PAIRS = _N_CHUNKS // 2


def _sc_gather(data, table):
    mesh = plsc.VectorSubcoreMesh(core_axis_name="c", subcore_axis_name="s")

    @functools.partial(
        pl.kernel,
        mesh=mesh,
        out_type=jax.ShapeDtypeStruct((_R, _S, _D), jnp.float32),
        scratch_types=[
            pltpu.VMEM((_ROWS_W, _S), jnp.int32),
            pltpu.VMEM((_ROWS_C, _S, _D), jnp.float32),
            pltpu.VMEM((_ROWS_C, _S, _D), jnp.float32),
            pltpu.SemaphoreType.DMA,
            pltpu.SemaphoreType.DMA,
            pltpu.SemaphoreType.DMA,
            pltpu.SemaphoreType.DMA,
        ],
        compiler_params=pltpu.CompilerParams(use_tc_tiling_on_sc=False),
    )
    def k(idx_hbm, table_hbm, out_hbm, idx_all, rows0, rows1, sg0, sg1, sw0, sw1):
        wid = lax.axis_index("s") * _NC + lax.axis_index("c")
        base = wid * _ROWS_W
        pltpu.sync_copy(idx_hbm.at[pl.ds(base, _ROWS_W)], idx_all)

        def gather(i, buf, sem):
            r0 = i * _ROWS_C
            for j in range(_ROWS_C):
                pltpu.async_copy(table_hbm.at[idx_all.at[r0 + j]], buf.at[j], sem)

        def wait_gather(i, buf, sem):
            r0 = i * _ROWS_C
            for j in range(_ROWS_C):
                pltpu.make_async_copy(
                    table_hbm.at[idx_all.at[r0 + j]], buf.at[j], sem
                ).wait()

        def write(i, buf, sem):
            pltpu.async_copy(buf, out_hbm.at[pl.ds(base + i * _ROWS_C, _ROWS_C)], sem)

        def wait_write(i, buf, sem):
            pltpu.make_async_copy(
                buf, out_hbm.at[pl.ds(base + i * _ROWS_C, _ROWS_C)], sem
            ).wait()

        gather(0, rows0, sg0)

        def body(g, carry):
            c0 = 2 * g
            c1 = c0 + 1
            wait_gather(c0, rows0, sg0)
            write(c0, rows0, sw0)

            @pl.when(g > 0)
            def _():
                wait_write(c0 - 1, rows1, sw1)

            gather(c1, rows1, sg1)
            wait_gather(c1, rows1, sg1)
            write(c1, rows1, sw1)
            wait_write(c0, rows0, sw0)

            @pl.when(g < _N_PAIRS - 1)
            def _():
                gather(c0 + 2, rows0, sg0)

            return carry

        lax.fori_loop(0, _N_PAIRS, body, 0)
        wait_write(_N_CHUNKS - 1, rows1, sw1)

    return k(data, table)


def kernel(data, embeddings):
    return _sc_gather(data, embeddings)


# speedup vs baseline: 1.3704x; 1.3704x over previous
import functools

import jax
import jax.numpy as jnp
from jax import lax
from jax.experimental import pallas as pl
from jax.experimental.pallas import tpu as pltpu
from jax.experimental.pallas import tpu_sc as plsc

_R, _S = 16384, 26
_V, _D = 10000, 64
_N = _R * _S
_L = 128
_NC, _NS = 2, 16
_NW = _NC * _NS
_PER_W = _N // _NW
_IROWS_W = _PER_W // _L
_IROWS_C = 4
_CHUNK = _IROWS_C * _L
_N_CHUNKS = _IROWS_W // _IROWS_C
_N_PAIRS = _N_CHUNKS // 2

_SD = _S * _D
_B1 = 512


def _sc_gather(idx2d, table):
    mesh = plsc.VectorSubcoreMesh(core_axis_name="c", subcore_axis_name="s")

    @functools.partial(
        pl.kernel,
        mesh=mesh,
        out_type=jax.ShapeDtypeStruct((_N, _D), jnp.float32),
        scratch_types=[
            pltpu.VMEM((_IROWS_W, _L), jnp.int32),
            pltpu.VMEM((_CHUNK, _D), jnp.float32),
            pltpu.VMEM((_CHUNK, _D), jnp.float32),
            pltpu.SemaphoreType.DMA,
            pltpu.SemaphoreType.DMA,
            pltpu.SemaphoreType.DMA,
            pltpu.SemaphoreType.DMA,
        ],
        compiler_params=pltpu.CompilerParams(use_tc_tiling_on_sc=False),
    )
    def k(idx_hbm, table_hbm, out_hbm, idx_all, rows0, rows1, sg0, sg1, sw0, sw1):
        wid = lax.axis_index("s") * _NC + lax.axis_index("c")
        base = wid * _PER_W
        pltpu.sync_copy(idx_hbm.at[pl.ds(wid * _IROWS_W, _IROWS_W)], idx_all)

        def gather(c, buf, sem):
            for j in range(_IROWS_C):
                pltpu.async_copy(
                    table_hbm.at[idx_all.at[c * _IROWS_C + j]],
                    buf.at[pl.ds(j * _L, _L)],
                    sem,
                )

        def wait_gather(c, buf, sem):
            for j in range(_IROWS_C):
                pltpu.make_async_copy(
                    table_hbm.at[idx_all.at[c * _IROWS_C + j]],
                    buf.at[pl.ds(j * _L, _L)],
                    sem,
                ).wait()

        def write(c, buf, sem):
            pltpu.async_copy(buf, out_hbm.at[pl.ds(base + c * _CHUNK, _CHUNK)], sem)

        def wait_write(c, buf, sem):
            pltpu.make_async_copy(
                buf, out_hbm.at[pl.ds(base + c * _CHUNK, _CHUNK)], sem
            ).wait()

        gather(0, rows0, sg0)

        def body(g, carry):
            c0 = 2 * g
            c1 = c0 + 1
            wait_gather(c0, rows0, sg0)
            write(c0, rows0, sw0)

            @pl.when(g > 0)
            def _():
                wait_write(c0 - 1, rows1, sw1)

            gather(c1, rows1, sg1)
            wait_gather(c1, rows1, sg1)
            write(c1, rows1, sw1)
            wait_write(c0, rows0, sw0)

            @pl.when(g < _N_PAIRS - 1)
            def _():
                gather(c0 + 2, rows0, sg0)

            return carry

        lax.fori_loop(0, _N_PAIRS, body, 0)
        wait_write(_N_CHUNKS - 1, rows1, sw1)

    return k(idx2d, table)


def _tc_transpose_kernel(x_ref, y_ref):
    y_ref[...] = x_ref[...].T


def _tc_transpose(x):
    return pl.pallas_call(
        _tc_transpose_kernel,
        grid=(_R // _B1,),
        in_specs=[pl.BlockSpec((_B1, _SD), lambda i: (i, 0))],
        out_specs=pl.BlockSpec((_SD, _B1), lambda i: (0, i)),
        out_shape=jax.ShapeDtypeStruct((_SD, _R), jnp.float32),
    )(x)


def kernel(data, embeddings):
    idx2d = data.reshape(_N // _L, _L)
    flat = _sc_gather(idx2d, embeddings)
    y = _tc_transpose(flat.reshape(_R, _SD))
    return jnp.transpose(y.reshape(_S, _D, _R), (2, 0, 1))
